# baseline (device time: 249105 ns/iter reference)
import functools

import jax
import jax.numpy as jnp
from jax import lax
from jax.experimental import pallas as pl
from jax.experimental.pallas import tpu as pltpu

N_DEV = 16
N_TOK = 8192
TOK_PER = 512
D = 256
H = 512
E_PER = 4
CAP = 102


def kernel(x, router_W, route_idx, expert_W):
    del router_W

    def body(x_ref, route_ref, w_ref, out_ref,
             xall, routeall, mask_ref, contrib, rs_send, rs_recv,
             agx_ssem, agx_rsem, agr_ssem, agr_rsem, rs_ssem, rs_rsem):
        j = lax.axis_index("i")
        left = lax.rem(j - 1 + N_DEV, N_DEV)
        right = lax.rem(j + 1, N_DEV)

        barrier = pltpu.get_barrier_semaphore()
        for nbr in (left, right):
            pl.semaphore_signal(barrier, inc=1, device_id=(nbr,),
                                device_id_type=pl.DeviceIdType.MESH)
        pl.semaphore_wait(barrier, 2)

        xall[pl.ds(j * TOK_PER, TOK_PER), :] = x_ref[:, :].astype(jnp.bfloat16)
        routeall[pl.ds(j * TOK_PER, TOK_PER), :] = route_ref[:, :]

        for h in range(N_DEV - 1):
            c = lax.rem(j - h + N_DEV, N_DEV)
            row = c * TOK_PER
            rx = pltpu.make_async_remote_copy(
                src_ref=xall.at[pl.ds(row, TOK_PER), :],
                dst_ref=xall.at[pl.ds(row, TOK_PER), :],
                send_sem=agx_ssem.at[h],
                recv_sem=agx_rsem.at[h],
                device_id=(right,),
                device_id_type=pl.DeviceIdType.MESH,
            )
            rr = pltpu.make_async_remote_copy(
                src_ref=routeall.at[pl.ds(row, TOK_PER), :],
                dst_ref=routeall.at[pl.ds(row, TOK_PER), :],
                send_sem=agr_ssem.at[h],
                recv_sem=agr_rsem.at[h],
                device_id=(right,),
                device_id_type=pl.DeviceIdType.MESH,
            )
            rx.start()
            rr.start()
            rx.wait()
            rr.wait()

        e0 = j * E_PER
        eids = e0 + lax.broadcasted_iota(jnp.int32, (1, E_PER), 1)
        onehot = (routeall[:, :] == eids).astype(jnp.float32)
        rowi = lax.broadcasted_iota(jnp.int32, (TOK_PER, TOK_PER), 0)
        coli = lax.broadcasted_iota(jnp.int32, (TOK_PER, TOK_PER), 1)
        tril = (rowi >= coli).astype(jnp.float32)
        base = jnp.zeros((1, E_PER), jnp.float32)
        for b in range(N_DEV):
            ob = onehot[b * TOK_PER:(b + 1) * TOK_PER, :]
            cb = jnp.dot(tril, ob, preferred_element_type=jnp.float32) + base
            keep = jnp.logical_and(ob > 0, cb <= CAP)
            mask_ref[b * TOK_PER:(b + 1) * TOK_PER, :] = jnp.where(
                keep, 1.0, 0.0).astype(jnp.bfloat16)
            base = base + jnp.sum(ob, axis=0, keepdims=True)

        wks = [w_ref[k, :, :].astype(jnp.bfloat16) for k in range(E_PER)]

        def chunk_body(b, carry):
            row = b * TOK_PER
            xc = xall[pl.ds(row, TOK_PER), :]
            mc = mask_ref[pl.ds(row, TOK_PER), :]
            acc = jnp.zeros((TOK_PER, H), jnp.float32)
            for k in range(E_PER):
                acc = acc + jnp.dot(xc * mc[:, k:k + 1], wks[k],
                                    preferred_element_type=jnp.float32)
            contrib[pl.ds(row, TOK_PER), :] = acc.astype(jnp.bfloat16)
            return carry

        lax.fori_loop(0, N_DEV, chunk_body, 0)

        for t in range(N_DEV - 1):
            c = lax.rem(j - 1 - t + 2 * N_DEV, N_DEV)
            row = c * TOK_PER
            chunk = contrib[pl.ds(row, TOK_PER), :]
            if t == 0:
                rs_send[:, :] = chunk
            else:
                rs_send[:, :] = chunk + rs_recv[t - 1, :, :]
            rdma = pltpu.make_async_remote_copy(
                src_ref=rs_send,
                dst_ref=rs_recv.at[t],
                send_sem=rs_ssem.at[t],
                recv_sem=rs_rsem.at[t],
                device_id=(right,),
                device_id_type=pl.DeviceIdType.MESH,
            )
            rdma.start()
            rdma.wait()

        out_ref[:, :] = (contrib[pl.ds(j * TOK_PER, TOK_PER), :]
                         + rs_recv[N_DEV - 2, :, :]).astype(jnp.float32)

        @functools.partial(pl.run_scoped, sem=pltpu.SemaphoreType.REGULAR)
        def _(sem):
            for nbr in (left, right):
                pl.semaphore_signal(sem, inc=1, device_id=(nbr,),
                                    device_id_type=pl.DeviceIdType.MESH)
            pl.semaphore_wait(sem, 2)

    return pl.pallas_call(
        body,
        out_shape=jax.ShapeDtypeStruct((TOK_PER, H), jnp.float32),
        in_specs=[
            pl.BlockSpec(memory_space=pltpu.VMEM),
            pl.BlockSpec(memory_space=pltpu.VMEM),
            pl.BlockSpec(memory_space=pltpu.VMEM),
        ],
        out_specs=pl.BlockSpec(memory_space=pltpu.VMEM),
        scratch_shapes=[
            pltpu.VMEM((N_TOK, D), jnp.bfloat16),
            pltpu.VMEM((N_TOK, 1), jnp.int32),
            pltpu.VMEM((N_TOK, E_PER), jnp.bfloat16),
            pltpu.VMEM((N_TOK, H), jnp.bfloat16),
            pltpu.VMEM((TOK_PER, H), jnp.bfloat16),
            pltpu.VMEM((N_DEV - 1, TOK_PER, H), jnp.bfloat16),
            pltpu.SemaphoreType.DMA((N_DEV - 1,)),
            pltpu.SemaphoreType.DMA((N_DEV - 1,)),
            pltpu.SemaphoreType.DMA((N_DEV - 1,)),
            pltpu.SemaphoreType.DMA((N_DEV - 1,)),
            pltpu.SemaphoreType.DMA((N_DEV - 1,)),
            pltpu.SemaphoreType.DMA((N_DEV - 1,)),
        ],
        compiler_params=pltpu.CompilerParams(collective_id=0),
    )(x, route_idx, expert_W)


# device time: 212520 ns/iter; 1.1721x vs baseline; 1.1721x over previous
import functools

import jax
import jax.numpy as jnp
from jax import lax
from jax.experimental import pallas as pl
from jax.experimental.pallas import tpu as pltpu

N_DEV = 16
N_TOK = 8192
TOK_PER = 512
D = 256
H = 512
E_PER = 4
CAP = 102


def kernel(x, router_W, route_idx, expert_W):
    del router_W

    def body(x_ref, route_ref, w_ref, out_ref,
             xall, routeall, bases_ref, own_contrib, rs_send, rs_recv,
             agx_ssem, agx_rsem, agr_ssem, agr_rsem, rs_ssem, rs_rsem):
        j = lax.axis_index("i")
        left = lax.rem(j - 1 + N_DEV, N_DEV)
        right = lax.rem(j + 1, N_DEV)

        barrier = pltpu.get_barrier_semaphore()
        for nbr in (left, right):
            pl.semaphore_signal(barrier, inc=1, device_id=(nbr,),
                                device_id_type=pl.DeviceIdType.MESH)
        pl.semaphore_wait(barrier, 2)

        xall[pl.ds(j * TOK_PER, TOK_PER), :] = x_ref[:, :].astype(jnp.bfloat16)
        routeall[pl.ds(j * TOK_PER, TOK_PER), :] = route_ref[:, :]

        def ag_rdma(ref, c, ssem, rsem, h):
            row = c * TOK_PER
            return pltpu.make_async_remote_copy(
                src_ref=ref.at[pl.ds(row, TOK_PER), :],
                dst_ref=ref.at[pl.ds(row, TOK_PER), :],
                send_sem=ssem.at[h],
                recv_sem=rsem.at[h],
                device_id=(right,),
                device_id_type=pl.DeviceIdType.MESH,
            )

        drain = []

        x_rdmas = [ag_rdma(xall, j, agx_ssem, agx_rsem, 0)]
        x_rdmas[0].start()

        for h in range(N_DEV - 1):
            c = lax.rem(j - h + N_DEV, N_DEV)
            rr = ag_rdma(routeall, c, agr_ssem, agr_rsem, h)
            if h > 0:
                drain[-1].wait_recv()
            rr.start()
            drain.append(rr)
        drain[-1].wait_recv()

        e0 = j * E_PER
        eids = e0 + lax.broadcasted_iota(jnp.int32, (1, E_PER), 1)
        running = jnp.zeros((1, E_PER), jnp.float32)
        for b in range(N_DEV):
            oh_b = (routeall[b * TOK_PER:(b + 1) * TOK_PER, :]
                    == eids).astype(jnp.float32)
            bases_ref[pl.ds(b, 1), :] = running
            running = running + jnp.sum(oh_b, axis=0, keepdims=True)

        rowi = lax.broadcasted_iota(jnp.int32, (TOK_PER, TOK_PER), 0)
        coli = lax.broadcasted_iota(jnp.int32, (TOK_PER, TOK_PER), 1)
        tril = (rowi >= coli).astype(jnp.float32)
        wks = [w_ref[k, :, :].astype(jnp.bfloat16) for k in range(E_PER)]

        def chunk_contrib(c):
            row = c * TOK_PER
            rc = routeall[pl.ds(row, TOK_PER), :]
            oh = (rc == eids).astype(jnp.float32)
            rank = jnp.dot(tril, oh, preferred_element_type=jnp.float32)
            rank = rank + bases_ref[pl.ds(c, 1), :]
            mc = jnp.where(jnp.logical_and(oh > 0, rank <= CAP),
                           1.0, 0.0).astype(jnp.bfloat16)
            xc = xall[pl.ds(row, TOK_PER), :]
            acc = jnp.zeros((TOK_PER, H), jnp.float32)
            for k in range(E_PER):
                acc = acc + jnp.dot(xc * mc[:, k:k + 1], wks[k],
                                    preferred_element_type=jnp.float32)
            return acc

        rs_rdmas = []
        for t in range(1, N_DEV):
            x_rdmas[t - 1].wait_recv()
            if t < N_DEV - 1:
                rx = ag_rdma(xall, lax.rem(j - t + N_DEV, N_DEV),
                             agx_ssem, agx_rsem, t)
                rx.start()
                x_rdmas.append(rx)
            c = lax.rem(j - t + 2 * N_DEV, N_DEV)
            acc = chunk_contrib(c)
            s = t - 1
            if s >= 1:
                rs_rdmas[s - 1].wait_recv()
                acc = acc + rs_recv[s - 1, :, :].astype(jnp.float32)
            if s >= 2:
                rs_rdmas[s - 2].wait_send()
            rs_send[s % 2, :, :] = acc.astype(jnp.bfloat16)
            rs = pltpu.make_async_remote_copy(
                src_ref=rs_send.at[s % 2],
                dst_ref=rs_recv.at[s],
                send_sem=rs_ssem.at[s],
                recv_sem=rs_rsem.at[s],
                device_id=(right,),
                device_id_type=pl.DeviceIdType.MESH,
            )
            rs.start()
            rs_rdmas.append(rs)

        own_contrib[:, :] = chunk_contrib(j)

        rs_rdmas[N_DEV - 2].wait_recv()
        out_ref[:, :] = own_contrib[:, :] + rs_recv[N_DEV - 2, :, :].astype(
            jnp.float32)

        for r in x_rdmas + drain + rs_rdmas[N_DEV - 3:]:
            r.wait_send()

        @functools.partial(pl.run_scoped, sem=pltpu.SemaphoreType.REGULAR)
        def _(sem):
            for nbr in (left, right):
                pl.semaphore_signal(sem, inc=1, device_id=(nbr,),
                                    device_id_type=pl.DeviceIdType.MESH)
            pl.semaphore_wait(sem, 2)

    return pl.pallas_call(
        body,
        out_shape=jax.ShapeDtypeStruct((TOK_PER, H), jnp.float32),
        in_specs=[
            pl.BlockSpec(memory_space=pltpu.VMEM),
            pl.BlockSpec(memory_space=pltpu.VMEM),
            pl.BlockSpec(memory_space=pltpu.VMEM),
        ],
        out_specs=pl.BlockSpec(memory_space=pltpu.VMEM),
        scratch_shapes=[
            pltpu.VMEM((N_TOK, D), jnp.bfloat16),
            pltpu.VMEM((N_TOK, 1), jnp.int32),
            pltpu.VMEM((N_DEV, E_PER), jnp.float32),
            pltpu.VMEM((TOK_PER, H), jnp.float32),
            pltpu.VMEM((2, TOK_PER, H), jnp.bfloat16),
            pltpu.VMEM((N_DEV - 1, TOK_PER, H), jnp.bfloat16),
            pltpu.SemaphoreType.DMA((N_DEV - 1,)),
            pltpu.SemaphoreType.DMA((N_DEV - 1,)),
            pltpu.SemaphoreType.DMA((N_DEV - 1,)),
            pltpu.SemaphoreType.DMA((N_DEV - 1,)),
            pltpu.SemaphoreType.DMA((N_DEV - 1,)),
            pltpu.SemaphoreType.DMA((N_DEV - 1,)),
        ],
        compiler_params=pltpu.CompilerParams(collective_id=0),
    )(x, route_idx, expert_W)
